# Initial kernel scaffold; baseline (speedup 1.0000x reference)
import jax
import jax.numpy as jnp
from jax.experimental import pallas as pl


def kernel(points, features, bn_fts_g, bn_fts_b, ec1_w0, ec1_w1, ec1_w2, ec1_g0, ec1_g1, ec1_g2, ec1_b0, ec1_b1, ec1_b2, ec2_w0, ec2_w1, ec2_w2, ec2_g0, ec2_g1, ec2_g2, ec2_b0, ec2_b1, ec2_b2, ec2_sc_w, ec2_sc_g, ec2_sc_b, fus_w, fus_g, fus_b, fc0_w, fc0_b, fc1_w, fc1_b):
    raise NotImplementedError("write your pallas kernel here")


# speedup vs baseline: 8.0401x; 8.0401x over previous
import functools

import jax
import jax.numpy as jnp
from jax.experimental import pallas as pl

_EPS = 1e-5
_K = 7


def _pairwise(P):
    N = P.shape[0]
    xx = jnp.sum(P * P, axis=1, keepdims=True)
    ones = jnp.ones((N, 1), jnp.float32)
    A = jnp.concatenate([2.0 * P, -ones, -xx], axis=1)
    Bm = jnp.concatenate([P, xx, ones], axis=1)
    return jax.lax.dot_general(
        A, Bm, (((1,), (1,)), ((), ())),
        preferred_element_type=jnp.float32,
        precision=jax.lax.Precision.HIGHEST)


def _edge_block(P, F, Wxd, Wd, b0, W1, b1, W2, b2, cout):
    N = P.shape[0]
    pd = _pairwise(P)
    A = jnp.dot(F, Wxd, preferred_element_type=jnp.float32) + b0
    H = jnp.dot(F, Wd, preferred_element_type=jnp.float32)
    colid = jax.lax.broadcasted_iota(jnp.int32, (N, N), 1)

    def select(pd):
        m = jnp.max(pd, axis=1, keepdims=True)
        cand = jnp.where(pd == m, colid, N)
        sel = jnp.min(cand, axis=1, keepdims=True)
        return colid == sel

    oh0 = select(pd)
    pd = jnp.where(oh0, -jnp.inf, pd)

    def body(_, carry):
        pd, acc = carry
        oh = select(pd)
        pd = jnp.where(oh, -jnp.inf, pd)
        G = jnp.dot(oh.astype(jnp.float32), H,
                    preferred_element_type=jnp.float32)
        Y = jax.nn.relu(A + G)
        Y = jax.nn.relu(jnp.dot(Y, W1, preferred_element_type=jnp.float32) + b1)
        Y = jax.nn.relu(jnp.dot(Y, W2, preferred_element_type=jnp.float32) + b2)
        return pd, acc + Y

    acc0 = jnp.zeros((N, cout), jnp.float32)
    _, acc = jax.lax.fori_loop(0, _K, body, (pd, acc0))
    return acc * (1.0 / _K)


def _body(pts_ref, fts_ref,
          sfts_ref, bfts_ref,
          e1xd_ref, e1d_ref, e1b0_ref, e1w1_ref, e1b1_ref, e1w2_ref, e1b2_ref,
          e2xd_ref, e2d_ref, e2b0_ref, e2w1_ref, e2b1_ref, e2w2_ref, e2b2_ref,
          wsc_ref, bsc_ref, wfus_ref, bfus_ref,
          wfc0_ref, bfc0_ref, wfc1_ref, bfc1_ref,
          out_ref):
    Pt = pts_ref[0]
    F0 = fts_ref[0]
    mask = (jnp.sum(jnp.abs(F0), axis=1, keepdims=True) != 0.0)
    mask = mask.astype(jnp.float32)
    shift = (1.0 - mask) * 1e9
    F = (F0 * sfts_ref[...] + bfts_ref[...]) * mask

    P1 = Pt * mask + shift
    m1 = _edge_block(P1, F, e1xd_ref[...], e1d_ref[...], e1b0_ref[...],
                     e1w1_ref[...], e1b1_ref[...], e1w2_ref[...], e1b2_ref[...],
                     32)
    F1 = jax.nn.relu(F + m1) * mask

    P2 = F1 + shift
    m2 = _edge_block(P2, F1, e2xd_ref[...], e2d_ref[...], e2b0_ref[...],
                     e2w1_ref[...], e2b1_ref[...], e2w2_ref[...], e2b2_ref[...],
                     64)
    sc = jnp.dot(F1, wsc_ref[...], preferred_element_type=jnp.float32) \
        + bsc_ref[...]
    F2 = jax.nn.relu(sc + m2) * mask

    Fc = jnp.concatenate([F1, F2], axis=1)
    Yf = jax.nn.relu(
        jnp.dot(Fc, wfus_ref[...], preferred_element_type=jnp.float32)
        + bfus_ref[...]) * mask
    counts = jnp.maximum(jnp.sum(mask), 1.0)
    pooled = jnp.sum(Yf, axis=0, keepdims=True) / counts
    h = jax.nn.relu(
        jnp.dot(pooled, wfc0_ref[...], preferred_element_type=jnp.float32)
        + bfc0_ref[...])
    out_ref[0] = jnp.dot(h, wfc1_ref[...],
                         preferred_element_type=jnp.float32) + bfc1_ref[...]


def kernel(points, features, bn_fts_g, bn_fts_b,
           ec1_w0, ec1_w1, ec1_w2, ec1_g0, ec1_g1, ec1_g2,
           ec1_b0, ec1_b1, ec1_b2,
           ec2_w0, ec2_w1, ec2_w2, ec2_g0, ec2_g1, ec2_g2,
           ec2_b0, ec2_b1, ec2_b2,
           ec2_sc_w, ec2_sc_g, ec2_sc_b,
           fus_w, fus_g, fus_b, fc0_w, fc0_b, fc1_w, fc1_b):
    B, _, N = features.shape
    rs = 1.0 / jnp.sqrt(jnp.float32(1.0 + _EPS))

    def fold(W, g, b):
        return ((g * rs)[:, None] * W).T, b[None, :]

    e1w0, e1b0 = fold(ec1_w0, ec1_g0, ec1_b0)
    e1w1, e1b1 = fold(ec1_w1, ec1_g1, ec1_b1)
    e1w2, e1b2 = fold(ec1_w2, ec1_g2, ec1_b2)
    e2w0, e2b0 = fold(ec2_w0, ec2_g0, ec2_b0)
    e2w1, e2b1 = fold(ec2_w1, ec2_g1, ec2_b1)
    e2w2, e2b2 = fold(ec2_w2, ec2_g2, ec2_b2)
    wsc, bsc = fold(ec2_sc_w, ec2_sc_g, ec2_sc_b)
    wfus, bfus = fold(fus_w, fus_g, fus_b)
    e1xd, e1d = e1w0[:32] - e1w0[32:], e1w0[32:]
    e2xd, e2d = e2w0[:32] - e2w0[32:], e2w0[32:]
    sfts = (bn_fts_g * rs)[None, :]
    bfts = bn_fts_b[None, :]
    wfc0, bfc0 = fc0_w.T, fc0_b[None, :]
    wfc1, bfc1 = fc1_w.T, fc1_b[None, :]

    pts_t = jnp.transpose(points, (0, 2, 1))
    fts_t = jnp.transpose(features, (0, 2, 1))

    def bspec(shape):
        return pl.BlockSpec(shape, lambda b: (0,) * len(shape))

    ws = [sfts, bfts,
          e1xd, e1d, e1b0, e1w1, e1b1, e1w2, e1b2,
          e2xd, e2d, e2b0, e2w1, e2b1, e2w2, e2b2,
          wsc, bsc, wfus, bfus, wfc0, bfc0, wfc1, bfc1]
    in_specs = [pl.BlockSpec((1, N, 2), lambda b: (b, 0, 0)),
                pl.BlockSpec((1, N, 32), lambda b: (b, 0, 0))]
    in_specs += [bspec(w.shape) for w in ws]

    out = pl.pallas_call(
        _body,
        grid=(B,),
        in_specs=in_specs,
        out_specs=pl.BlockSpec((1, 1, 10), lambda b: (b, 0, 0)),
        out_shape=jax.ShapeDtypeStruct((B, 1, 10), jnp.float32),
    )(pts_t, fts_t, *ws)
    return out.reshape(B, 10)
